# baseline (speedup 1.0000x reference)
import jax
import jax.numpy as jnp
from jax.experimental import pallas as pl

_NUM_EXPERTS = 8
_HIDDEN = 768
_BR = 2048


def _gate_kernel(x_ref, w_ref, idx_ref, wgt_ref):
    x = x_ref[...]
    w = w_ref[...]
    logits = jnp.dot(x, w, preferred_element_type=jnp.float32)

    iota = jax.lax.broadcasted_iota(jnp.int32, logits.shape, 1)
    m = jnp.max(logits, axis=1, keepdims=True)
    e = jnp.exp(logits - m)
    s = jnp.sum(e, axis=1, keepdims=True)

    i1 = jnp.min(jnp.where(logits == m, iota, _NUM_EXPERTS), axis=1, keepdims=True)
    masked = jnp.where(iota == i1, -jnp.inf, logits)
    v2 = jnp.max(masked, axis=1, keepdims=True)
    i2 = jnp.min(jnp.where(masked == v2, iota, _NUM_EXPERTS), axis=1, keepdims=True)

    p1 = 1.0 / s
    p2 = jnp.exp(v2 - m) / s

    idx_ref[...] = jnp.concatenate([i1, i2], axis=1).astype(jnp.int32)
    wgt_ref[...] = jnp.concatenate([p1, p2], axis=1)


def kernel(x, weight):
    b, s, h = x.shape
    n = b * s
    xs = x.reshape(n, h)
    wt = weight.T

    grid = (n // _BR,)
    idx, wgt = pl.pallas_call(
        _gate_kernel,
        grid=grid,
        in_specs=[
            pl.BlockSpec((_BR, h), lambda i: (i, 0)),
            pl.BlockSpec((h, _NUM_EXPERTS), lambda i: (0, 0)),
        ],
        out_specs=[
            pl.BlockSpec((_BR, 2), lambda i: (i, 0)),
            pl.BlockSpec((_BR, 2), lambda i: (i, 0)),
        ],
        out_shape=[
            jax.ShapeDtypeStruct((n, 2), jnp.int32),
            jax.ShapeDtypeStruct((n, 2), jnp.float32),
        ],
    )(xs, wt)
    return (idx, wgt)


# speedup vs baseline: 2.1395x; 2.1395x over previous
import jax
import jax.numpy as jnp
from jax.experimental import pallas as pl

_NUM_EXPERTS = 8
_HIDDEN = 768
_BR = 2048


def _gate_kernel(x_ref, w_ref, idx_ref, wgt_ref):
    x = x_ref[...]
    w = w_ref[...]
    logits = jax.lax.dot_general(
        w, x, (((1,), (1,)), ((), ())),
        preferred_element_type=jnp.float32,
    )

    iota_e = jax.lax.broadcasted_iota(jnp.int32, logits.shape, 0)
    m = jnp.max(logits, axis=0, keepdims=True)
    e = jnp.exp(logits - m)
    s = jnp.sum(e, axis=0, keepdims=True)

    i1 = jnp.min(jnp.where(logits == m, iota_e, _NUM_EXPERTS), axis=0, keepdims=True)
    masked = jnp.where(iota_e == i1, -jnp.inf, logits)
    v2 = jnp.max(masked, axis=0, keepdims=True)
    i2 = jnp.min(jnp.where(masked == v2, iota_e, _NUM_EXPERTS), axis=0, keepdims=True)

    p1 = 1.0 / s
    p2 = jnp.exp(v2 - m) / s

    idx_ref[...] = jnp.concatenate([i1, i2], axis=0)
    wgt_ref[...] = jnp.concatenate([p1, p2], axis=0)


def kernel(x, weight):
    b, s, h = x.shape
    n = b * s
    xs = x.reshape(n, h)

    grid = (n // _BR,)
    idx_t, wgt_t = pl.pallas_call(
        _gate_kernel,
        grid=grid,
        in_specs=[
            pl.BlockSpec((_BR, h), lambda i: (i, 0)),
            pl.BlockSpec((_NUM_EXPERTS, h), lambda i: (0, 0)),
        ],
        out_specs=[
            pl.BlockSpec((2, _BR), lambda i: (0, i)),
            pl.BlockSpec((2, _BR), lambda i: (0, i)),
        ],
        out_shape=[
            jax.ShapeDtypeStruct((2, n), jnp.int32),
            jax.ShapeDtypeStruct((2, n), jnp.float32),
        ],
    )(xs, weight)
    return (idx_t.T, wgt_t.T)


# speedup vs baseline: 2.1979x; 1.0273x over previous
import jax
import jax.numpy as jnp
from jax.experimental import pallas as pl
from jax.experimental.pallas import tpu as pltpu

_NUM_EXPERTS = 8
_HIDDEN = 768
_BR = 4096


def _gate_kernel(x_ref, w_ref, idx_ref, wgt_ref):
    x = x_ref[...]
    w = w_ref[...]
    logits = jax.lax.dot_general(
        w, x, (((1,), (1,)), ((), ())),
        preferred_element_type=jnp.float32,
    )

    iota_e = jax.lax.broadcasted_iota(jnp.int32, logits.shape, 0)
    m = jnp.max(logits, axis=0, keepdims=True)
    e = jnp.exp(logits - m)
    s = jnp.sum(e, axis=0, keepdims=True)

    i1 = jnp.min(jnp.where(logits == m, iota_e, _NUM_EXPERTS), axis=0, keepdims=True)
    masked = jnp.where(iota_e == i1, -jnp.inf, logits)
    v2 = jnp.max(masked, axis=0, keepdims=True)
    i2 = jnp.min(jnp.where(masked == v2, iota_e, _NUM_EXPERTS), axis=0, keepdims=True)

    p1 = 1.0 / s
    p2 = jnp.exp(v2 - m) / s

    idx_ref[...] = jnp.concatenate([i1, i2], axis=0)
    wgt_ref[...] = jnp.concatenate([p1, p2], axis=0)


def kernel(x, weight):
    b, s, h = x.shape
    n = b * s
    xs = x.reshape(n, h)

    grid = (n // _BR,)
    idx_t, wgt_t = pl.pallas_call(
        _gate_kernel,
        grid=grid,
        in_specs=[
            pl.BlockSpec((_BR, h), lambda i: (i, 0)),
            pl.BlockSpec((_NUM_EXPERTS, h), lambda i: (0, 0)),
        ],
        out_specs=[
            pl.BlockSpec((2, _BR), lambda i: (0, i)),
            pl.BlockSpec((2, _BR), lambda i: (0, i)),
        ],
        out_shape=[
            jax.ShapeDtypeStruct((2, n), jnp.int32),
            jax.ShapeDtypeStruct((2, n), jnp.float32),
        ],
        compiler_params=pltpu.CompilerParams(
            dimension_semantics=("parallel",),
        ),
    )(xs, weight)
    return (idx_t.T, wgt_t.T)
